# Initial kernel scaffold; baseline (speedup 1.0000x reference)
import jax
import jax.numpy as jnp
from jax.experimental import pallas as pl


def kernel(points_tensor, batch_atributes, W, b):
    raise NotImplementedError("write your pallas kernel here")


# speedup vs baseline: 74.1363x; 74.1363x over previous
import jax
import jax.numpy as jnp
from jax.experimental import pallas as pl

C_IN = 16
C_OUT = 32
KSIZE = 2
NUM_CELLS = KSIZE * 8
RADIUS = 0.2
_CELL_W = RADIUS / KSIZE

_R = 256


def _body(pts_all_ref, pts_row_ref, attrs_ref, w_ref, b_ref, out_ref):
    pts_all = pts_all_ref[0]
    pts_row = pts_row_ref[0]
    ax = attrs_ref[0]

    n = pts_all.shape[1]
    r = pts_row.shape[1]

    dx = pts_row[0].reshape(r, 1) - pts_all[0].reshape(1, n)
    dy = pts_row[1].reshape(r, 1) - pts_all[1].reshape(1, n)
    dz = pts_row[2].reshape(r, 1) - pts_all[2].reshape(1, n)
    dist = jnp.sqrt(dx * dx + dy * dy + dz * dz + jnp.float32(1e-12))

    shell = jnp.clip(jnp.floor(dist / jnp.float32(_CELL_W)), 0.0, float(KSIZE - 1))
    octant = (jnp.where(dx >= 0, 4.0, 0.0)
              + jnp.where(dy >= 0, 2.0, 0.0)
              + jnp.where(dz >= 0, 1.0, 0.0))
    cell = shell * 8.0 + octant
    cell = jnp.where(dist < jnp.float32(RADIUS), cell, float(NUM_CELLS))

    cols = []
    for q in range(NUM_CELLS):
        mask = jnp.where(cell == float(q), 1.0, 0.0)
        sc = jax.lax.dot_general(
            mask, ax, (((1,), (0,)), ((), ())),
            preferred_element_type=jnp.float32,
            precision=jax.lax.Precision.HIGHEST)
        cnt = jnp.maximum(sc[:, C_IN:C_IN + 1], 1.0)
        cols.append(sc[:, :C_IN] / cnt)
    g = jnp.concatenate(cols, axis=1)

    out = jax.lax.dot_general(
        g, w_ref[...], (((1,), (0,)), ((), ())),
        preferred_element_type=jnp.float32,
        precision=jax.lax.Precision.HIGHEST) + b_ref[0]
    out_ref[0] = out


def kernel(points_tensor, batch_atributes, W, b):
    B, N, _ = points_tensor.shape
    pts_t = jnp.transpose(points_tensor, (0, 2, 1))
    ones = jnp.ones((B, N, 1), dtype=jnp.float32)
    attrs_ext = jnp.concatenate([batch_atributes, ones], axis=-1)
    w_flat = jnp.transpose(W, (2, 1, 0)).reshape(NUM_CELLS * C_IN, C_OUT)
    b2 = b.reshape(1, C_OUT)

    grid = (B, N // _R)
    return pl.pallas_call(
        _body,
        grid=grid,
        in_specs=[
            pl.BlockSpec((1, 3, N), lambda bb, rb: (bb, 0, 0)),
            pl.BlockSpec((1, 3, _R), lambda bb, rb: (bb, 0, rb)),
            pl.BlockSpec((1, N, C_IN + 1), lambda bb, rb: (bb, 0, 0)),
            pl.BlockSpec((NUM_CELLS * C_IN, C_OUT), lambda bb, rb: (0, 0)),
            pl.BlockSpec((1, C_OUT), lambda bb, rb: (0, 0)),
        ],
        out_specs=pl.BlockSpec((1, _R, C_OUT), lambda bb, rb: (bb, rb, 0)),
        out_shape=jax.ShapeDtypeStruct((B, N, C_OUT), jnp.float32),
    )(pts_t, pts_t, attrs_ext, w_flat, b2)


# speedup vs baseline: 291.1377x; 3.9271x over previous
import jax
import jax.numpy as jnp
from jax.experimental import pallas as pl

C_IN = 16
C_OUT = 32
KSIZE = 2
NUM_CELLS = KSIZE * 8
RADIUS = 0.2
_CELL_W = RADIUS / KSIZE

_R = 256


def _body(pts_all_ref, pts_row_ref, attrs_ref, w_ref, b_ref, out_ref):
    pts_all = pts_all_ref[0]
    pts_row = pts_row_ref[0]
    ax = attrs_ref[0]

    n = pts_all.shape[1]
    r = pts_row.shape[1]

    dx = pts_row[0].reshape(r, 1) - pts_all[0].reshape(1, n)
    dy = pts_row[1].reshape(r, 1) - pts_all[1].reshape(1, n)
    dz = pts_row[2].reshape(r, 1) - pts_all[2].reshape(1, n)
    d2e = dx * dx + dy * dy + dz * dz + jnp.float32(1e-12)

    shell = jnp.where(d2e >= jnp.float32(_CELL_W * _CELL_W), 8.0, 0.0)
    octant = (jnp.where(dx >= 0, 4.0, 0.0)
              + jnp.where(dy >= 0, 2.0, 0.0)
              + jnp.where(dz >= 0, 1.0, 0.0))
    cell = shell + octant
    cell = jnp.where(d2e < jnp.float32(RADIUS * RADIUS), cell, float(NUM_CELLS))

    one_b = jnp.bfloat16(1.0)
    zero_b = jnp.bfloat16(0.0)
    cell_b = cell.astype(jnp.bfloat16)
    cols = []
    for q in range(NUM_CELLS):
        mask = jnp.where(cell_b == jnp.bfloat16(q), one_b, zero_b)
        sc = jax.lax.dot_general(
            mask, ax, (((1,), (0,)), ((), ())),
            preferred_element_type=jnp.float32)
        cnt = jnp.maximum(sc[:, C_IN:C_IN + 1], 1.0)
        cols.append(sc[:, :C_IN] / cnt)
    g = jnp.concatenate(cols, axis=1)

    out = jax.lax.dot_general(
        g, w_ref[...], (((1,), (0,)), ((), ())),
        preferred_element_type=jnp.float32,
        precision=jax.lax.Precision.HIGHEST) + b_ref[0]
    out_ref[0] = out


def kernel(points_tensor, batch_atributes, W, b):
    B, N, _ = points_tensor.shape
    pts_t = jnp.transpose(points_tensor, (0, 2, 1))
    ones = jnp.ones((B, N, 1), dtype=jnp.float32)
    attrs_ext = jnp.concatenate([batch_atributes, ones], axis=-1).astype(jnp.bfloat16)
    w_flat = jnp.transpose(W, (2, 1, 0)).reshape(NUM_CELLS * C_IN, C_OUT)
    b2 = b.reshape(1, C_OUT)

    grid = (B, N // _R)
    return pl.pallas_call(
        _body,
        grid=grid,
        in_specs=[
            pl.BlockSpec((1, 3, N), lambda bb, rb: (bb, 0, 0)),
            pl.BlockSpec((1, 3, _R), lambda bb, rb: (bb, 0, rb)),
            pl.BlockSpec((1, N, C_IN + 1), lambda bb, rb: (bb, 0, 0)),
            pl.BlockSpec((NUM_CELLS * C_IN, C_OUT), lambda bb, rb: (0, 0)),
            pl.BlockSpec((1, C_OUT), lambda bb, rb: (0, 0)),
        ],
        out_specs=pl.BlockSpec((1, _R, C_OUT), lambda bb, rb: (bb, rb, 0)),
        out_shape=jax.ShapeDtypeStruct((B, N, C_OUT), jnp.float32),
    )(pts_t, pts_t, attrs_ext, w_flat, b2)
